# baseline (speedup 1.0000x reference)
import jax
import jax.numpy as jnp
from jax import lax
from jax.experimental import pallas as pl
from jax.experimental.pallas import tpu as pltpu
from jax.experimental.pallas import tpu_sc as plsc

MARGIN = 0.2
K = 8
N = 4096
NW = 32
ROWS_PER_W = N // NW
NVREG = N // 16
CHUNK = 8
NCHUNK = ROWS_PER_W // CHUNK


def _sort16(x, descending=False):
    if descending:
        return -lax.sort(-x)
    return lax.sort(x)


def _merge(a_asc, b_desc):
    return jnp.maximum(a_asc, b_desc)


def _merge_tree_kv(items, descending):
    if len(items) == 1:
        return plsc.sort_key_val(items[0][0], items[0][1],
                                 descending=descending)
    h = len(items) // 2
    av, ai = _merge_tree_kv(items[:h], False)
    bv, bi = _merge_tree_kv(items[h:], True)
    mv = jnp.maximum(av, bv)
    mi = jnp.where(av >= bv, ai, bi)
    return plsc.sort_key_val(mv, mi, descending=descending)


def _merge_tree_v(items, descending):
    if len(items) == 1:
        return _sort16(items[0], descending)
    h = len(items) // 2
    m = _merge(_merge_tree_v(items[:h], False), _merge_tree_v(items[h:], True))
    return _sort16(m, descending)


def _body(logits_hbm, out_hbm, buf_a, buf_b, accv, sem_a, sem_b):
    f32 = jnp.float32
    i32 = jnp.int32
    cid = lax.axis_index("c")
    sid = lax.axis_index("s")
    wid = sid * 2 + cid
    base = wid * ROWS_PER_W

    lane = lax.iota(i32, 16)
    lane0 = lane == 0
    topmask = lane < K
    neg_inf = jnp.full((16,), -jnp.inf, f32)

    def row_topk(i, buf, off):
        i_vec = jnp.full((16,), i, i32) + off
        p_vec = plsc.load_gather(buf, [i_vec])
        plsc.store_scatter(buf, [i_vec], neg_inf, mask=lane0)

        items = []
        for g in range(16):
            gm = buf[pl.ds(off + 256 * g, 16)]
            for k in range(1, 16):
                gm = jnp.maximum(gm, buf[pl.ds(off + 256 * g + 16 * k, 16)])
            items.append((gm, 16 * g + lane))

        _, fi = _merge_tree_kv(items, True)

        dn = lax.GatherDimensionNumbers(
            offset_dims=(), collapsed_slice_dims=(0,), start_index_map=(0,))
        vecs = []
        for k in range(K):
            bc = lax.gather(fi, jnp.full((16, 1), k, i32), dn,
                            slice_sizes=(1,),
                            mode=lax.GatherScatterMode.PROMISE_IN_BOUNDS)
            col = ((bc >> 4) << 8) + (bc & 15) + 16 * lane + off
            vecs.append(plsc.load_gather(buf, [col]))

        f = _merge_tree_v(vecs, True)

        vals = jnp.maximum(f - p_vec + MARGIN, 0.0)
        return jnp.where(topmask, vals, 0.0)

    def fire_chunk(c, buf, sem):
        row0 = base + CHUNK * c
        for r in range(CHUNK):
            pltpu.async_copy(logits_hbm.at[row0 + r],
                             buf.at[pl.ds(r * N, N)], sem)

    def drain_chunk(buf, sem):
        for r in range(CHUNK):
            pltpu.make_async_copy(logits_hbm.at[base],
                                  buf.at[pl.ds(r * N, N)], sem).wait()

    def compute_chunk(c, buf, acc):
        row0 = base + CHUNK * c

        def rbody(rr, acc):
            r = 2 * rr
            c0 = row_topk(row0 + r, buf, r * N)
            c1 = row_topk(row0 + r + 1, buf, (r + 1) * N)
            return acc + (c0 + c1)

        return lax.fori_loop(0, CHUNK // 2, rbody, acc)

    fire_chunk(0, buf_a, sem_a)

    def pair_body(cc, acc):
        c0 = 2 * cc
        c1 = c0 + 1
        drain_chunk(buf_a, sem_a)
        fire_chunk(c1, buf_b, sem_b)
        acc = compute_chunk(c0, buf_a, acc)
        drain_chunk(buf_b, sem_b)
        fire_chunk(jnp.minimum(c1 + 1, NCHUNK - 1), buf_a, sem_a)
        acc = compute_chunk(c1, buf_b, acc)
        return acc

    acc = lax.fori_loop(0, NCHUNK // 2, pair_body, jnp.zeros((16,), f32))
    drain_chunk(buf_a, sem_a)

    accv[...] = acc
    pltpu.sync_copy(accv, out_hbm.at[wid])


def kernel(logits):
    mesh = plsc.VectorSubcoreMesh(core_axis_name="c", subcore_axis_name="s")
    out = pl.kernel(
        _body,
        out_type=jax.ShapeDtypeStruct((NW, 16), jnp.float32),
        name="triplet_topk_sc",
        mesh=mesh,
        scratch_types=[
            pltpu.VMEM((CHUNK * N,), jnp.float32),
            pltpu.VMEM((CHUNK * N,), jnp.float32),
            pltpu.VMEM((16,), jnp.float32),
            pltpu.SemaphoreType.DMA,
            pltpu.SemaphoreType.DMA,
        ],
        compiler_params=pltpu.CompilerParams(needs_layout_passes=False),
    )(logits)
    return jnp.sum(out) / N


# speedup vs baseline: 1.2180x; 1.2180x over previous
import jax
import jax.numpy as jnp
from jax import lax
from jax.experimental import pallas as pl
from jax.experimental.pallas import tpu as pltpu
from jax.experimental.pallas import tpu_sc as plsc

MARGIN = 0.2
K = 8
N = 4096
NW = 32
ROWS_PER_W = N // NW
NVREG = N // 16
CHUNK = 8
NCHUNK = ROWS_PER_W // CHUNK


def _sort16(x, descending=False):
    if descending:
        return -lax.sort(-x)
    return lax.sort(x)


def _merge(a_asc, b_desc):
    return jnp.maximum(a_asc, b_desc)


def _merge_tree_kv(items, descending):
    if len(items) == 1:
        return plsc.sort_key_val(items[0][0], items[0][1],
                                 descending=descending)
    h = len(items) // 2
    av, ai = _merge_tree_kv(items[:h], False)
    bv, bi = _merge_tree_kv(items[h:], True)
    mv = jnp.maximum(av, bv)
    mi = jnp.where(av >= bv, ai, bi)
    return plsc.sort_key_val(mv, mi, descending=descending)


def _merge_tree_v(items, descending):
    if len(items) == 1:
        return _sort16(items[0], descending)
    h = len(items) // 2
    m = _merge(_merge_tree_v(items[:h], False), _merge_tree_v(items[h:], True))
    return _sort16(m, descending)


def _body(logits_hbm, out_hbm, buf_a, buf_b, accv, sem_a, sem_b):
    f32 = jnp.float32
    i32 = jnp.int32
    cid = lax.axis_index("c")
    sid = lax.axis_index("s")
    wid = sid * 2 + cid
    base = wid * ROWS_PER_W

    lane = lax.iota(i32, 16)
    lane0 = lane == 0
    topmask = lane < K
    neg_inf = jnp.full((16,), -jnp.inf, f32)

    def row_topk(i, buf, off, acc):
        i_vec = jnp.full((16,), i, i32) + off
        p_vec = plsc.load_gather(buf, [i_vec])
        plsc.store_scatter(buf, [i_vec], neg_inf, mask=lane0)

        items = []
        for g in range(16):
            gm = buf[pl.ds(off + 256 * g, 16)]
            for k in range(1, 16):
                gm = jnp.maximum(gm, buf[pl.ds(off + 256 * g + 16 * k, 16)])
            items.append((gm, 16 * g + lane))

        _, fi = _merge_tree_kv(items, True)

        dn = lax.GatherDimensionNumbers(
            offset_dims=(), collapsed_slice_dims=(0,), start_index_map=(0,))
        vecs = []
        for k in range(K):
            bc = lax.gather(fi, jnp.full((16, 1), k, i32), dn,
                            slice_sizes=(1,),
                            mode=lax.GatherScatterMode.PROMISE_IN_BOUNDS)
            col = ((bc >> 4) << 8) + (bc & 15) + 16 * lane + off
            vecs.append(plsc.load_gather(buf, [col]))

        f = _merge_tree_v(vecs, True)

        vals = jnp.maximum(f - p_vec + MARGIN, 0.0)
        return acc + jnp.where(topmask, vals, 0.0)

    def fire_chunk(c, buf, sem):
        row0 = base + CHUNK * c
        for r in range(CHUNK):
            pltpu.async_copy(logits_hbm.at[row0 + r],
                             buf.at[pl.ds(r * N, N)], sem)

    def drain_chunk(buf, sem):
        for r in range(CHUNK):
            pltpu.make_async_copy(logits_hbm.at[base],
                                  buf.at[pl.ds(r * N, N)], sem).wait()

    def compute_drain(c, buf, sem, acc):
        row0 = base + CHUNK * c

        def rbody(r, acc):
            pltpu.make_async_copy(logits_hbm.at[base],
                                  buf.at[pl.ds(r * N, N)], sem).wait()
            return row_topk(row0 + r, buf, r * N, acc)

        return lax.fori_loop(0, CHUNK, rbody, acc)

    fire_chunk(0, buf_a, sem_a)

    def pair_body(cc, acc):
        c0 = 2 * cc
        c1 = c0 + 1
        fire_chunk(c1, buf_b, sem_b)
        acc = compute_drain(c0, buf_a, sem_a, acc)
        fire_chunk(jnp.minimum(c1 + 1, NCHUNK - 1), buf_a, sem_a)
        acc = compute_drain(c1, buf_b, sem_b, acc)
        return acc

    acc = lax.fori_loop(0, NCHUNK // 2, pair_body, jnp.zeros((16,), f32))
    drain_chunk(buf_a, sem_a)

    accv[...] = acc
    pltpu.sync_copy(accv, out_hbm.at[wid])


def kernel(logits):
    mesh = plsc.VectorSubcoreMesh(core_axis_name="c", subcore_axis_name="s")
    out = pl.kernel(
        _body,
        out_type=jax.ShapeDtypeStruct((NW, 16), jnp.float32),
        name="triplet_topk_sc",
        mesh=mesh,
        scratch_types=[
            pltpu.VMEM((CHUNK * N,), jnp.float32),
            pltpu.VMEM((CHUNK * N,), jnp.float32),
            pltpu.VMEM((16,), jnp.float32),
            pltpu.SemaphoreType.DMA,
            pltpu.SemaphoreType.DMA,
        ],
        compiler_params=pltpu.CompilerParams(needs_layout_passes=False),
    )(logits)
    return jnp.sum(out) / N
